# baseline (speedup 1.0000x reference)
import functools

import jax
import jax.numpy as jnp
from jax import lax
from jax.experimental import pallas as pl
from jax.experimental.pallas import tpu as pltpu
from jax.experimental.pallas import tpu_sc as plsc

_B = 4096
_EMB = 64
_HID = (512, 256, 128)
_TEMP = 0.1
_BN_INV = float(1.0 / (1.0 + 1e-5) ** 0.5)


def _sc_gather8(idxs, tabs):
    info = plsc.get_sparse_core_info()
    nc, ns = info.num_cores, info.num_subcores
    nw = nc * ns
    bpw = _B // nw

    mesh = plsc.VectorSubcoreMesh(core_axis_name="c", subcore_axis_name="s")

    @functools.partial(
        pl.kernel,
        mesh=mesh,
        out_type=tuple(
            jax.ShapeDtypeStruct((_B, _EMB), jnp.float32) for _ in range(8)
        ),
        scratch_types=[
            pltpu.VMEM((bpw,), jnp.int32),
            pltpu.VMEM((bpw, _EMB), jnp.float32),
            pltpu.SemaphoreType.DMA,
        ],
        compiler_params=pltpu.CompilerParams(use_tc_tiling_on_sc=False),
    )
    def gather_kernel(*refs):
        idx_refs = refs[0:8]
        tab_refs = refs[8:16]
        out_refs = refs[16:24]
        idx_v, rows_v, sem = refs[24], refs[25], refs[26]
        wid = lax.axis_index("s") * nc + lax.axis_index("c")
        base = wid * bpw
        for f in range(8):
            pltpu.sync_copy(idx_refs[f].at[pl.ds(base, bpw)], idx_v)
            pltpu.async_copy(tab_refs[f].at[idx_v], rows_v, sem).wait()
            pltpu.sync_copy(rows_v, out_refs[f].at[pl.ds(base, bpw)])

    return gather_kernel(*idxs, *tabs)


_T_BLK = 1024


def _tower_block(e_refs, w_refs):
    w0, b0, g0, bt0, w1, b1, g1, bt1, w2, b2, g2, bt2 = w_refs
    x = None
    for f in range(4):
        part = jnp.dot(
            e_refs[f][...],
            w0[f * _EMB:(f + 1) * _EMB, :],
            preferred_element_type=jnp.float32,
        )
        x = part if x is None else x + part
    x = jnp.maximum(x + b0[...], 0.0)
    x = (g0[...] * _BN_INV) * x + bt0[...]
    x = jnp.dot(x, w1[...], preferred_element_type=jnp.float32)
    x = jnp.maximum(x + b1[...], 0.0)
    x = (g1[...] * _BN_INV) * x + bt1[...]
    x = jnp.dot(x, w2[...], preferred_element_type=jnp.float32)
    x = jnp.maximum(x + b2[...], 0.0)
    x = (g2[...] * _BN_INV) * x + bt2[...]
    nrm = jnp.sqrt(jnp.sum(x * x, axis=-1, keepdims=True))
    return x / jnp.maximum(nrm, 1e-12)


def _towers_kernel(*refs):
    eu = refs[0:4]
    ei = refs[4:8]
    wu = refs[8:20]
    wi = refs[20:32]
    ue_ref, ie_ref = refs[32], refs[33]
    ue_ref[...] = _tower_block(eu, wu)
    ie_ref[...] = _tower_block(ei, wi)


def _towers_tc(eu, ei, wu, wi):
    nblk = _B // _T_BLK
    e_spec = pl.BlockSpec((_T_BLK, _EMB), lambda i: (i, 0))

    def _full(a):
        nd = a.ndim
        return pl.BlockSpec(a.shape, lambda i, _n=nd: (0,) * _n)

    in_specs = (
        [e_spec] * 8
        + [_full(a) for a in wu]
        + [_full(a) for a in wi]
    )
    out_spec = pl.BlockSpec((_T_BLK, _HID[-1]), lambda i: (i, 0))
    out_shape = (
        jax.ShapeDtypeStruct((_B, _HID[-1]), jnp.float32),
        jax.ShapeDtypeStruct((_B, _HID[-1]), jnp.float32),
    )
    return pl.pallas_call(
        _towers_kernel,
        grid=(nblk,),
        in_specs=in_specs,
        out_specs=(out_spec, out_spec),
        out_shape=out_shape,
    )(*eu, *ei, *wu, *wi)


_L_BLK = 512


def _logits_kernel(ue_ref, ie_ref, out_ref):
    out_ref[...] = lax.dot_general(
        ue_ref[...],
        ie_ref[...],
        (((1,), (1,)), ((), ())),
        preferred_element_type=jnp.float32,
    ) * (1.0 / _TEMP)


def _logits_tc(ue, ie):
    nblk = _B // _L_BLK
    return pl.pallas_call(
        _logits_kernel,
        grid=(nblk,),
        in_specs=[
            pl.BlockSpec((_L_BLK, _HID[-1]), lambda i: (i, 0)),
            pl.BlockSpec((_B, _HID[-1]), lambda i: (0, 0)),
        ],
        out_specs=pl.BlockSpec((_L_BLK, _B), lambda i: (i, 0)),
        out_shape=jax.ShapeDtypeStruct((_B, _B), jnp.float32),
    )(ue, ie)


def kernel(
    user_id, emb_user_id,
    user_age, emb_user_age,
    user_gender, emb_user_gender,
    user_region, emb_user_region,
    item_id, emb_item_id,
    item_category, emb_item_category,
    item_brand, emb_item_brand,
    item_price_bucket, emb_item_price_bucket,
    u_W0, u_b0, u_g0, u_beta0,
    u_W1, u_b1, u_g1, u_beta1,
    u_W2, u_b2, u_g2, u_beta2,
    i_W0, i_b0, i_g0, i_beta0,
    i_W1, i_b1, i_g1, i_beta1,
    i_W2, i_b2, i_g2, i_beta2,
):
    idxs = [
        jnp.asarray(a, jnp.int32)
        for a in (user_id, user_age, user_gender, user_region,
                  item_id, item_category, item_brand, item_price_bucket)
    ]
    tabs = [emb_user_id, emb_user_age, emb_user_gender, emb_user_region,
            emb_item_id, emb_item_category, emb_item_brand,
            emb_item_price_bucket]
    gathered = _sc_gather8(idxs, tabs)
    eu, ei = gathered[0:4], gathered[4:8]

    def _prep(b, g, bt):
        return (b.reshape(1, -1), g.reshape(1, -1), bt.reshape(1, -1))

    wu = (u_W0, *_prep(u_b0, u_g0, u_beta0),
          u_W1, *_prep(u_b1, u_g1, u_beta1),
          u_W2, *_prep(u_b2, u_g2, u_beta2))
    wi = (i_W0, *_prep(i_b0, i_g0, i_beta0),
          i_W1, *_prep(i_b1, i_g1, i_beta1),
          i_W2, *_prep(i_b2, i_g2, i_beta2))

    ue, ie = _towers_tc(eu, ei, wu, wi)
    return _logits_tc(ue, ie)


# speedup vs baseline: 1.0406x; 1.0406x over previous
import functools

import jax
import jax.numpy as jnp
from jax import lax
from jax.experimental import pallas as pl
from jax.experimental.pallas import tpu as pltpu
from jax.experimental.pallas import tpu_sc as plsc

_B = 4096
_EMB = 64
_HID = (512, 256, 128)
_TEMP = 0.1
_BN_INV = float(1.0 / (1.0 + 1e-5) ** 0.5)


def _sc_gather8(idxs, tabs):
    info = plsc.get_sparse_core_info()
    nc, ns = info.num_cores, info.num_subcores
    nw = nc * ns
    bpw = _B // nw

    mesh = plsc.VectorSubcoreMesh(core_axis_name="c", subcore_axis_name="s")

    @functools.partial(
        pl.kernel,
        mesh=mesh,
        out_type=tuple(
            jax.ShapeDtypeStruct((_B, _EMB), jnp.float32) for _ in range(8)
        ),
        scratch_types=[
            pltpu.VMEM((8, bpw), jnp.int32),
            pltpu.VMEM((8, bpw, _EMB), jnp.float32),
            pltpu.SemaphoreType.DMA,
            pltpu.SemaphoreType.DMA,
            pltpu.SemaphoreType.DMA,
        ],
        compiler_params=pltpu.CompilerParams(use_tc_tiling_on_sc=False),
    )
    def gather_kernel(*refs):
        idx_refs = refs[0:8]
        tab_refs = refs[8:16]
        out_refs = refs[16:24]
        idx_b, rows_b = refs[24], refs[25]
        sem_i, sem_g, sem_w = refs[26], refs[27], refs[28]
        wid = lax.axis_index("s") * nc + lax.axis_index("c")
        base = wid * bpw

        stages = [
            pltpu.async_copy(
                idx_refs[f].at[pl.ds(base, bpw)], idx_b.at[f], sem_i)
            for f in range(8)
        ]
        for c in stages:
            c.wait()
        gathers = [
            pltpu.async_copy(
                tab_refs[f].at[idx_b.at[f]], rows_b.at[f], sem_g)
            for f in range(8)
        ]
        for c in gathers:
            c.wait()
        writes = [
            pltpu.async_copy(
                rows_b.at[f], out_refs[f].at[pl.ds(base, bpw)], sem_w)
            for f in range(8)
        ]
        for c in writes:
            c.wait()

    return gather_kernel(*idxs, *tabs)


_T_BLK = 1024


def _tower_block(e_refs, w_refs):
    w0, b0, g0, bt0, w1, b1, g1, bt1, w2, b2, g2, bt2 = w_refs
    x = None
    for f in range(4):
        part = jnp.dot(
            e_refs[f][...],
            w0[f * _EMB:(f + 1) * _EMB, :],
            preferred_element_type=jnp.float32,
        )
        x = part if x is None else x + part
    x = jnp.maximum(x + b0[...], 0.0)
    x = (g0[...] * _BN_INV) * x + bt0[...]
    x = jnp.dot(x, w1[...], preferred_element_type=jnp.float32)
    x = jnp.maximum(x + b1[...], 0.0)
    x = (g1[...] * _BN_INV) * x + bt1[...]
    x = jnp.dot(x, w2[...], preferred_element_type=jnp.float32)
    x = jnp.maximum(x + b2[...], 0.0)
    x = (g2[...] * _BN_INV) * x + bt2[...]
    nrm = jnp.sqrt(jnp.sum(x * x, axis=-1, keepdims=True))
    return x / jnp.maximum(nrm, 1e-12)


def _towers_kernel(*refs):
    eu = refs[0:4]
    ei = refs[4:8]
    wu = refs[8:20]
    wi = refs[20:32]
    ue_ref, ie_ref = refs[32], refs[33]
    ue_ref[...] = _tower_block(eu, wu)
    ie_ref[...] = _tower_block(ei, wi)


def _towers_tc(eu, ei, wu, wi):
    nblk = _B // _T_BLK
    e_spec = pl.BlockSpec((_T_BLK, _EMB), lambda i: (i, 0))

    def _full(a):
        nd = a.ndim
        return pl.BlockSpec(a.shape, lambda i, _n=nd: (0,) * _n)

    in_specs = (
        [e_spec] * 8
        + [_full(a) for a in wu]
        + [_full(a) for a in wi]
    )
    out_spec = pl.BlockSpec((_T_BLK, _HID[-1]), lambda i: (i, 0))
    out_shape = (
        jax.ShapeDtypeStruct((_B, _HID[-1]), jnp.float32),
        jax.ShapeDtypeStruct((_B, _HID[-1]), jnp.float32),
    )
    return pl.pallas_call(
        _towers_kernel,
        grid=(nblk,),
        in_specs=in_specs,
        out_specs=(out_spec, out_spec),
        out_shape=out_shape,
    )(*eu, *ei, *wu, *wi)


_L_BLK = 512


def _logits_kernel(ue_ref, ie_ref, out_ref):
    out_ref[...] = lax.dot_general(
        ue_ref[...],
        ie_ref[...],
        (((1,), (1,)), ((), ())),
        preferred_element_type=jnp.float32,
    ) * (1.0 / _TEMP)


def _logits_tc(ue, ie):
    nblk = _B // _L_BLK
    return pl.pallas_call(
        _logits_kernel,
        grid=(nblk,),
        in_specs=[
            pl.BlockSpec((_L_BLK, _HID[-1]), lambda i: (i, 0)),
            pl.BlockSpec((_B, _HID[-1]), lambda i: (0, 0)),
        ],
        out_specs=pl.BlockSpec((_L_BLK, _B), lambda i: (i, 0)),
        out_shape=jax.ShapeDtypeStruct((_B, _B), jnp.float32),
    )(ue, ie)


def kernel(
    user_id, emb_user_id,
    user_age, emb_user_age,
    user_gender, emb_user_gender,
    user_region, emb_user_region,
    item_id, emb_item_id,
    item_category, emb_item_category,
    item_brand, emb_item_brand,
    item_price_bucket, emb_item_price_bucket,
    u_W0, u_b0, u_g0, u_beta0,
    u_W1, u_b1, u_g1, u_beta1,
    u_W2, u_b2, u_g2, u_beta2,
    i_W0, i_b0, i_g0, i_beta0,
    i_W1, i_b1, i_g1, i_beta1,
    i_W2, i_b2, i_g2, i_beta2,
):
    idxs = [
        jnp.asarray(a, jnp.int32)
        for a in (user_id, user_age, user_gender, user_region,
                  item_id, item_category, item_brand, item_price_bucket)
    ]
    tabs = [emb_user_id, emb_user_age, emb_user_gender, emb_user_region,
            emb_item_id, emb_item_category, emb_item_brand,
            emb_item_price_bucket]
    gathered = _sc_gather8(idxs, tabs)
    eu, ei = gathered[0:4], gathered[4:8]

    def _prep(b, g, bt):
        return (b.reshape(1, -1), g.reshape(1, -1), bt.reshape(1, -1))

    wu = (u_W0, *_prep(u_b0, u_g0, u_beta0),
          u_W1, *_prep(u_b1, u_g1, u_beta1),
          u_W2, *_prep(u_b2, u_g2, u_beta2))
    wi = (i_W0, *_prep(i_b0, i_g0, i_beta0),
          i_W1, *_prep(i_b1, i_g1, i_beta1),
          i_W2, *_prep(i_b2, i_g2, i_beta2))

    ue, ie = _towers_tc(eu, ei, wu, wi)
    return _logits_tc(ue, ie)
